# Initial kernel scaffold; baseline (speedup 1.0000x reference)
import jax
import jax.numpy as jnp
from jax.experimental import pallas as pl


def kernel(hidden_states, gate_w, w1, w2, w3):
    raise NotImplementedError("write your pallas kernel here")


# speedup vs baseline: 1.6737x; 1.6737x over previous
import functools

import jax
import jax.numpy as jnp
from jax import lax
from jax.experimental import pallas as pl
from jax.experimental.pallas import tpu as pltpu
from jax.experimental.pallas import tpu_sc as plsc

T = 8192
H = 1024
F = 3584
E = 8
BLK = 256
NT = (T * 2 + E * BLK) // BLK
P = NT * BLK
NC = 2
NS = 16
NW = NC * NS
PW = P // NW
GC = 96
TW = T // NW
CC = 32
NJ = 2
FC = F // NJ
RT = 1024


def _router_body(x_ref, g_ref, lg_ref, ti_ref, tw_ref):
    xb = x_ref[...]
    gw = g_ref[...]
    lg = lax.dot_general(xb, gw, (((1,), (1,)), ((), ())),
                         preferred_element_type=jnp.float32)
    lg_ref[...] = lg
    m = jnp.max(lg, axis=1, keepdims=True)
    p = jnp.exp(lg - m)
    p = p / jnp.sum(p, axis=1, keepdims=True)
    i1 = jnp.argmax(p, axis=1).astype(jnp.int32)
    v1 = jnp.max(p, axis=1)
    cols = lax.broadcasted_iota(jnp.int32, p.shape, 1)
    p2 = jnp.where(cols == i1[:, None], -1.0, p)
    i2 = jnp.argmax(p2, axis=1).astype(jnp.int32)
    v2 = jnp.max(p2, axis=1)
    s = v1 + v2
    ti_ref[...] = jnp.stack([i1, i2], axis=1)
    tw_ref[...] = jnp.stack([v1 / s, v2 / s], axis=1)


def _router(x, gate_w):
    return pl.pallas_call(
        _router_body,
        grid=(T // RT,),
        in_specs=[
            pl.BlockSpec((RT, H), lambda i: (i, 0)),
            pl.BlockSpec((E, H), lambda i: (0, 0)),
        ],
        out_specs=[
            pl.BlockSpec((RT, E), lambda i: (i, 0)),
            pl.BlockSpec((RT, 2), lambda i: (i, 0)),
            pl.BlockSpec((RT, 2), lambda i: (i, 0)),
        ],
        out_shape=[
            jax.ShapeDtypeStruct((T, E), jnp.float32),
            jax.ShapeDtypeStruct((T, 2), jnp.int32),
            jax.ShapeDtypeStruct((T, 2), jnp.float32),
        ],
    )(x, gate_w)


def _mlp_body(te_ref, xs_ref, w1_ref, w3_ref, w2_ref, wr_ref, out_ref, acc_ref):
    j = pl.program_id(1)
    xb = xs_ref[...]
    h1 = lax.dot_general(xb, w1_ref[0], (((1,), (1,)), ((), ())),
                         preferred_element_type=jnp.float32)
    h3 = lax.dot_general(xb, w3_ref[0], (((1,), (1,)), ((), ())),
                         preferred_element_type=jnp.float32)
    h = (h1 * jax.nn.sigmoid(h1)) * h3
    part = lax.dot_general(h, w2_ref[0], (((1,), (1,)), ((), ())),
                           preferred_element_type=jnp.float32)

    @pl.when(j == 0)
    def _():
        acc_ref[...] = part

    @pl.when(j != 0)
    def _():
        acc_ref[...] = acc_ref[...] + part

    @pl.when(j == NJ - 1)
    def _():
        out_ref[...] = acc_ref[...] * wr_ref[0, 0][:, None]


def _mlp(xs, w1, w3, w2, wrow3, te):
    return pl.pallas_call(
        _mlp_body,
        grid_spec=pltpu.PrefetchScalarGridSpec(
            num_scalar_prefetch=1,
            grid=(NT, NJ),
            in_specs=[
                pl.BlockSpec((BLK, H), lambda i, j, te_r: (i, 0)),
                pl.BlockSpec((1, FC, H), lambda i, j, te_r: (te_r[i], j, 0)),
                pl.BlockSpec((1, FC, H), lambda i, j, te_r: (te_r[i], j, 0)),
                pl.BlockSpec((1, H, FC), lambda i, j, te_r: (te_r[i], 0, j)),
                pl.BlockSpec((1, 1, BLK), lambda i, j, te_r: (i, 0, 0)),
            ],
            out_specs=pl.BlockSpec((BLK, H), lambda i, j, te_r: (i, 0)),
            scratch_shapes=[pltpu.VMEM((BLK, H), jnp.float32)],
        ),
        out_shape=jax.ShapeDtypeStruct((P, H), jnp.float32),
        compiler_params=pltpu.CompilerParams(
            dimension_semantics=("arbitrary", "arbitrary")),
    )(te, xs, w1, w3, w2, wrow3)


def _gather_body(x_hbm, rid_hbm, out_hbm, idx_v, rows_v, sem):
    wid = lax.axis_index("s") * NC + lax.axis_index("c")
    base = wid * PW
    pltpu.sync_copy(rid_hbm.at[pl.ds(base, PW)], idx_v)

    def chunk(c, _):
        cb = c * GC
        pltpu.async_copy(x_hbm.at[idx_v.at[pl.ds(cb, GC)]], rows_v, sem).wait()
        pltpu.sync_copy(rows_v, out_hbm.at[pl.ds(base + cb, GC)])
        return 0

    lax.fori_loop(0, PW // GC, chunk, 0)


def _gather(x, row_ids):
    mesh = plsc.VectorSubcoreMesh(core_axis_name="c", subcore_axis_name="s")
    f = functools.partial(
        pl.kernel,
        out_type=jax.ShapeDtypeStruct((P, H), jnp.float32),
        mesh=mesh,
        scratch_types=[
            pltpu.VMEM((PW,), jnp.int32),
            pltpu.VMEM((GC, H), jnp.float32),
            pltpu.SemaphoreType.DMA,
        ],
    )(_gather_body)
    return f(x, row_ids)


def _combine_body(ys_hbm, pa_hbm, pb_hbm, out_hbm, pa_v, pb_v, ra, rb, sem):
    wid = lax.axis_index("s") * NC + lax.axis_index("c")
    base = wid * TW
    pltpu.sync_copy(pa_hbm.at[pl.ds(base, TW)], pa_v)
    pltpu.sync_copy(pb_hbm.at[pl.ds(base, TW)], pb_v)

    def chunk(c, _):
        cb = c * CC
        pltpu.async_copy(ys_hbm.at[pa_v.at[pl.ds(cb, CC)]], ra, sem).wait()
        pltpu.async_copy(ys_hbm.at[pb_v.at[pl.ds(cb, CC)]], rb, sem).wait()

        def row(r, _):
            for q in range(H // 16):
                sl = pl.ds(q * 16, 16)
                ra[r, sl] = ra[r, sl] + rb[r, sl]
            return 0

        lax.fori_loop(0, CC, row, 0)
        pltpu.sync_copy(ra, out_hbm.at[pl.ds(base + cb, CC)])
        return 0

    lax.fori_loop(0, TW // CC, chunk, 0)


def _combine(ys, pos_a, pos_b):
    mesh = plsc.VectorSubcoreMesh(core_axis_name="c", subcore_axis_name="s")
    f = functools.partial(
        pl.kernel,
        out_type=jax.ShapeDtypeStruct((T, H), jnp.float32),
        mesh=mesh,
        scratch_types=[
            pltpu.VMEM((TW,), jnp.int32),
            pltpu.VMEM((TW,), jnp.int32),
            pltpu.VMEM((CC, H), jnp.float32),
            pltpu.VMEM((CC, H), jnp.float32),
            pltpu.SemaphoreType.DMA,
        ],
    )(_combine_body)
    return f(ys, pos_a, pos_b)


def kernel(hidden_states, gate_w, w1, w2, w3):
    bsz, seq, hd = hidden_states.shape
    x = hidden_states.reshape(-1, hd)

    logits, ti, tw = _router(x, gate_w)

    ef = ti.reshape(-1)
    wf = tw.reshape(-1)
    oh = (ef[:, None] == jnp.arange(E, dtype=jnp.int32)[None, :]).astype(jnp.int32)
    counts = oh.sum(axis=0)
    rank = jnp.take_along_axis(jnp.cumsum(oh, axis=0) - 1,
                               ef[:, None].astype(jnp.int32), axis=1)[:, 0]
    padded = ((counts + BLK - 1) // BLK) * BLK
    offs = jnp.concatenate(
        [jnp.zeros(1, jnp.int32), jnp.cumsum(padded).astype(jnp.int32)])[:E]
    pos = (offs[ef] + rank).astype(jnp.int32)
    tok = (jnp.arange(2 * T, dtype=jnp.int32) // 2)
    row_ids = jnp.zeros((P,), jnp.int32).at[pos].set(tok)
    wrow = jnp.zeros((P,), jnp.float32).at[pos].set(wf)
    tile_start = jnp.arange(NT, dtype=jnp.int32) * BLK
    te = jnp.clip(jnp.searchsorted(offs, tile_start, side="right") - 1,
                  0, E - 1).astype(jnp.int32)

    xs = _gather(x, row_ids)
    ys = _mlp(xs, w1, w3, w2, wrow.reshape(NT, 1, BLK), te)
    posr = pos.reshape(T, 2)
    final = _combine(ys, posr[:, 0], posr[:, 1])
    return final.reshape(bsz, seq, hd), logits


# speedup vs baseline: 2.0773x; 1.2411x over previous
import functools

import jax
import jax.numpy as jnp
from jax import lax
from jax.experimental import pallas as pl
from jax.experimental.pallas import tpu as pltpu
from jax.experimental.pallas import tpu_sc as plsc

T = 8192
H = 1024
F = 3584
E = 8
BLK = 256
NT = (T * 2 + E * BLK) // BLK
P = NT * BLK
NC = 2
NS = 16
NW = NC * NS
PW = P // NW
GC = 96
TW = T // NW
CC = 32
NJ = 2
FC = F // NJ
RT = 1024


def _router_body(x_ref, g_ref, lg_ref, ti_ref, tw_ref):
    xb = x_ref[...]
    gw = g_ref[...]
    lg = lax.dot_general(xb, gw, (((1,), (1,)), ((), ())),
                         preferred_element_type=jnp.float32)
    lg_ref[...] = lg
    m = jnp.max(lg, axis=1, keepdims=True)
    p = jnp.exp(lg - m)
    p = p / jnp.sum(p, axis=1, keepdims=True)
    i1 = jnp.argmax(p, axis=1).astype(jnp.int32)
    v1 = jnp.max(p, axis=1)
    cols = lax.broadcasted_iota(jnp.int32, p.shape, 1)
    p2 = jnp.where(cols == i1[:, None], -1.0, p)
    i2 = jnp.argmax(p2, axis=1).astype(jnp.int32)
    v2 = jnp.max(p2, axis=1)
    s = v1 + v2
    ti_ref[...] = jnp.stack([i1, i2], axis=1)
    tw_ref[...] = jnp.stack([v1 / s, v2 / s], axis=1)


def _router(x, gate_w):
    return pl.pallas_call(
        _router_body,
        grid=(T // RT,),
        in_specs=[
            pl.BlockSpec((RT, H), lambda i: (i, 0)),
            pl.BlockSpec((E, H), lambda i: (0, 0)),
        ],
        out_specs=[
            pl.BlockSpec((RT, E), lambda i: (i, 0)),
            pl.BlockSpec((RT, 2), lambda i: (i, 0)),
            pl.BlockSpec((RT, 2), lambda i: (i, 0)),
        ],
        out_shape=[
            jax.ShapeDtypeStruct((T, E), jnp.float32),
            jax.ShapeDtypeStruct((T, 2), jnp.int32),
            jax.ShapeDtypeStruct((T, 2), jnp.float32),
        ],
    )(x, gate_w)


def _mlp_body(te_ref, xs_ref, w1_ref, w3_ref, w2_ref, wr_ref, out_ref):
    xb = xs_ref[...].astype(jnp.bfloat16)
    h1 = lax.dot_general(xb, w1_ref[0], (((1,), (1,)), ((), ())),
                         preferred_element_type=jnp.float32)
    h3 = lax.dot_general(xb, w3_ref[0], (((1,), (1,)), ((), ())),
                         preferred_element_type=jnp.float32)
    h = ((h1 * jax.nn.sigmoid(h1)) * h3).astype(jnp.bfloat16)
    y = lax.dot_general(h, w2_ref[0], (((1,), (1,)), ((), ())),
                        preferred_element_type=jnp.float32)
    out_ref[...] = y * wr_ref[0, 0][:, None]


def _mlp(xs, w1, w3, w2, wrow3, te):
    return pl.pallas_call(
        _mlp_body,
        grid_spec=pltpu.PrefetchScalarGridSpec(
            num_scalar_prefetch=1,
            grid=(NT,),
            in_specs=[
                pl.BlockSpec((BLK, H), lambda i, te_r: (i, 0)),
                pl.BlockSpec((1, F, H), lambda i, te_r: (te_r[i], 0, 0)),
                pl.BlockSpec((1, F, H), lambda i, te_r: (te_r[i], 0, 0)),
                pl.BlockSpec((1, H, F), lambda i, te_r: (te_r[i], 0, 0)),
                pl.BlockSpec((1, 1, BLK), lambda i, te_r: (i, 0, 0)),
            ],
            out_specs=pl.BlockSpec((BLK, H), lambda i, te_r: (i, 0)),
        ),
        out_shape=jax.ShapeDtypeStruct((P, H), jnp.float32),
        compiler_params=pltpu.CompilerParams(
            dimension_semantics=("arbitrary",)),
    )(te, xs, w1, w3, w2, wrow3)


def _gather_body(x_hbm, rid_hbm, out_hbm, idx_v, rows_v, sem):
    wid = lax.axis_index("s") * NC + lax.axis_index("c")
    base = wid * PW
    pltpu.sync_copy(rid_hbm.at[pl.ds(base, PW)], idx_v)

    def chunk(c, _):
        cb = c * GC
        pltpu.async_copy(x_hbm.at[idx_v.at[pl.ds(cb, GC)]], rows_v, sem).wait()
        pltpu.sync_copy(rows_v, out_hbm.at[pl.ds(base + cb, GC)])
        return 0

    lax.fori_loop(0, PW // GC, chunk, 0)


def _gather(x, row_ids):
    mesh = plsc.VectorSubcoreMesh(core_axis_name="c", subcore_axis_name="s")
    f = functools.partial(
        pl.kernel,
        out_type=jax.ShapeDtypeStruct((P, H), jnp.float32),
        mesh=mesh,
        scratch_types=[
            pltpu.VMEM((PW,), jnp.int32),
            pltpu.VMEM((GC, H), jnp.float32),
            pltpu.SemaphoreType.DMA,
        ],
    )(_gather_body)
    return f(x, row_ids)


def _combine_body(ys_hbm, pa_hbm, pb_hbm, out_hbm, pa_v, pb_v, ra, rb, sem):
    wid = lax.axis_index("s") * NC + lax.axis_index("c")
    base = wid * TW
    pltpu.sync_copy(pa_hbm.at[pl.ds(base, TW)], pa_v)
    pltpu.sync_copy(pb_hbm.at[pl.ds(base, TW)], pb_v)

    def chunk(c, _):
        cb = c * CC
        pltpu.async_copy(ys_hbm.at[pa_v.at[pl.ds(cb, CC)]], ra, sem).wait()
        pltpu.async_copy(ys_hbm.at[pb_v.at[pl.ds(cb, CC)]], rb, sem).wait()

        def row(r, _):
            for q in range(H // 16):
                sl = pl.ds(q * 16, 16)
                ra[r, sl] = ra[r, sl] + rb[r, sl]
            return 0

        lax.fori_loop(0, CC, row, 0)
        pltpu.sync_copy(ra, out_hbm.at[pl.ds(base + cb, CC)])
        return 0

    lax.fori_loop(0, TW // CC, chunk, 0)


def _combine(ys, pos_a, pos_b):
    mesh = plsc.VectorSubcoreMesh(core_axis_name="c", subcore_axis_name="s")
    f = functools.partial(
        pl.kernel,
        out_type=jax.ShapeDtypeStruct((T, H), jnp.float32),
        mesh=mesh,
        scratch_types=[
            pltpu.VMEM((TW,), jnp.int32),
            pltpu.VMEM((TW,), jnp.int32),
            pltpu.VMEM((CC, H), jnp.float32),
            pltpu.VMEM((CC, H), jnp.float32),
            pltpu.SemaphoreType.DMA,
        ],
    )(_combine_body)
    return f(ys, pos_a, pos_b)


def kernel(hidden_states, gate_w, w1, w2, w3):
    bsz, seq, hd = hidden_states.shape
    x = hidden_states.reshape(-1, hd)

    logits, ti, tw = _router(x, gate_w)

    ef = ti.reshape(-1)
    wf = tw.reshape(-1)
    oh = (ef[:, None] == jnp.arange(E, dtype=jnp.int32)[None, :]).astype(jnp.int32)
    counts = oh.sum(axis=0)
    rank = jnp.take_along_axis(jnp.cumsum(oh, axis=0) - 1,
                               ef[:, None].astype(jnp.int32), axis=1)[:, 0]
    padded = ((counts + BLK - 1) // BLK) * BLK
    offs = jnp.concatenate(
        [jnp.zeros(1, jnp.int32), jnp.cumsum(padded).astype(jnp.int32)])[:E]
    pos = (offs[ef] + rank).astype(jnp.int32)
    tok = (jnp.arange(2 * T, dtype=jnp.int32) // 2)
    row_ids = jnp.zeros((P,), jnp.int32).at[pos].set(tok)
    wrow = jnp.zeros((P,), jnp.float32).at[pos].set(wf)
    tile_start = jnp.arange(NT, dtype=jnp.int32) * BLK
    te = jnp.clip(jnp.searchsorted(offs, tile_start, side="right") - 1,
                  0, E - 1).astype(jnp.int32)

    xs = _gather(x, row_ids)
    ys = _mlp(xs, w1.astype(jnp.bfloat16), w3.astype(jnp.bfloat16),
              w2.astype(jnp.bfloat16), wrow.reshape(NT, 1, BLK), te)
    posr = pos.reshape(T, 2)
    final = _combine(ys, posr[:, 0], posr[:, 1])
    return final.reshape(bsz, seq, hd), logits


# speedup vs baseline: 2.0927x; 1.0074x over previous
import functools

import jax
import jax.numpy as jnp
from jax import lax
from jax.experimental import pallas as pl
from jax.experimental.pallas import tpu as pltpu
from jax.experimental.pallas import tpu_sc as plsc

T = 8192
H = 1024
F = 3584
E = 8
BLK = 256
NT = (T * 2 + E * BLK) // BLK
P = NT * BLK
NC = 2
NS = 16
NW = NC * NS
PW = P // NW
GC = 48
TW = T // NW
CC = 32
NJ = 2
FC = F // NJ
RT = 1024


def _router_body(x_ref, g_ref, lg_ref, ti_ref, tw_ref):
    xb = x_ref[...]
    gw = g_ref[...]
    lg = lax.dot_general(xb, gw, (((1,), (1,)), ((), ())),
                         preferred_element_type=jnp.float32)
    lg_ref[...] = lg
    m = jnp.max(lg, axis=1, keepdims=True)
    p = jnp.exp(lg - m)
    p = p / jnp.sum(p, axis=1, keepdims=True)
    i1 = jnp.argmax(p, axis=1).astype(jnp.int32)
    v1 = jnp.max(p, axis=1)
    cols = lax.broadcasted_iota(jnp.int32, p.shape, 1)
    p2 = jnp.where(cols == i1[:, None], -1.0, p)
    i2 = jnp.argmax(p2, axis=1).astype(jnp.int32)
    v2 = jnp.max(p2, axis=1)
    s = v1 + v2
    ti_ref[...] = jnp.stack([i1, i2], axis=1)
    tw_ref[...] = jnp.stack([v1 / s, v2 / s], axis=1)


def _router(x, gate_w):
    return pl.pallas_call(
        _router_body,
        grid=(T // RT,),
        in_specs=[
            pl.BlockSpec((RT, H), lambda i: (i, 0)),
            pl.BlockSpec((E, H), lambda i: (0, 0)),
        ],
        out_specs=[
            pl.BlockSpec((RT, E), lambda i: (i, 0)),
            pl.BlockSpec((RT, 2), lambda i: (i, 0)),
            pl.BlockSpec((RT, 2), lambda i: (i, 0)),
        ],
        out_shape=[
            jax.ShapeDtypeStruct((T, E), jnp.float32),
            jax.ShapeDtypeStruct((T, 2), jnp.int32),
            jax.ShapeDtypeStruct((T, 2), jnp.float32),
        ],
    )(x, gate_w)


def _mlp_body(te_ref, xs_ref, w1_ref, w3_ref, w2_ref, wr_ref, out_ref):
    xb = xs_ref[...].astype(jnp.bfloat16)
    h1 = lax.dot_general(xb, w1_ref[0], (((1,), (1,)), ((), ())),
                         preferred_element_type=jnp.float32)
    h3 = lax.dot_general(xb, w3_ref[0], (((1,), (1,)), ((), ())),
                         preferred_element_type=jnp.float32)
    h = ((h1 * jax.nn.sigmoid(h1)) * h3).astype(jnp.bfloat16)
    y = lax.dot_general(h, w2_ref[0], (((1,), (1,)), ((), ())),
                        preferred_element_type=jnp.float32)
    out_ref[...] = y * wr_ref[0, 0][:, None]


def _mlp(xs, w1, w3, w2, wrow3, te):
    return pl.pallas_call(
        _mlp_body,
        grid_spec=pltpu.PrefetchScalarGridSpec(
            num_scalar_prefetch=1,
            grid=(NT,),
            in_specs=[
                pl.BlockSpec((BLK, H), lambda i, te_r: (i, 0)),
                pl.BlockSpec((1, F, H), lambda i, te_r: (te_r[i], 0, 0)),
                pl.BlockSpec((1, F, H), lambda i, te_r: (te_r[i], 0, 0)),
                pl.BlockSpec((1, H, F), lambda i, te_r: (te_r[i], 0, 0)),
                pl.BlockSpec((1, 1, BLK), lambda i, te_r: (i, 0, 0)),
            ],
            out_specs=pl.BlockSpec((BLK, H), lambda i, te_r: (i, 0)),
        ),
        out_shape=jax.ShapeDtypeStruct((P, H), jnp.float32),
        compiler_params=pltpu.CompilerParams(
            dimension_semantics=("arbitrary",)),
    )(te, xs, w1, w3, w2, wrow3)


def _gather_body(x_hbm, rid_hbm, out_hbm, idx_v, buf0, buf1, gs0, gs1, ss0, ss1):
    wid = lax.axis_index("s") * NC + lax.axis_index("c")
    base = wid * PW
    pltpu.sync_copy(rid_hbm.at[pl.ds(base, PW)], idx_v)
    n = PW // GC
    buf = (buf0, buf1)
    gsem = (gs0, gs1)
    ssem = (ss0, ss1)
    g = [None, None]
    s = [None, None]
    for c in range(n):
        b = c & 1
        if c == 0:
            g[b] = pltpu.async_copy(
                x_hbm.at[idx_v.at[pl.ds(0, GC)]], buf[b], gsem[b])
        g[b].wait()
        if c + 1 < n:
            nb = 1 - b
            if c >= 1:
                s[nb].wait()
            g[nb] = pltpu.async_copy(
                x_hbm.at[idx_v.at[pl.ds((c + 1) * GC, GC)]], buf[nb], gsem[nb])
        s[b] = pltpu.async_copy(
            buf[b], out_hbm.at[pl.ds(base + c * GC, GC)], ssem[b])
    s[(n - 1) & 1].wait()
    s[(n - 2) & 1].wait()


def _gather(x, row_ids):
    mesh = plsc.VectorSubcoreMesh(core_axis_name="c", subcore_axis_name="s")
    f = functools.partial(
        pl.kernel,
        out_type=jax.ShapeDtypeStruct((P, H), jnp.float32),
        mesh=mesh,
        scratch_types=[
            pltpu.VMEM((PW,), jnp.int32),
            pltpu.VMEM((GC, H), jnp.float32),
            pltpu.VMEM((GC, H), jnp.float32),
            pltpu.SemaphoreType.DMA,
            pltpu.SemaphoreType.DMA,
            pltpu.SemaphoreType.DMA,
            pltpu.SemaphoreType.DMA,
        ],
    )(_gather_body)
    return f(x, row_ids)


def _combine_body(ys_hbm, pa_hbm, pb_hbm, out_hbm, pa_v, pb_v, ra, rb, sem):
    wid = lax.axis_index("s") * NC + lax.axis_index("c")
    base = wid * TW
    pltpu.sync_copy(pa_hbm.at[pl.ds(base, TW)], pa_v)
    pltpu.sync_copy(pb_hbm.at[pl.ds(base, TW)], pb_v)

    def chunk(c, _):
        cb = c * CC
        pltpu.async_copy(ys_hbm.at[pa_v.at[pl.ds(cb, CC)]], ra, sem).wait()
        pltpu.async_copy(ys_hbm.at[pb_v.at[pl.ds(cb, CC)]], rb, sem).wait()

        def row(r, _):
            for q in range(H // 16):
                sl = pl.ds(q * 16, 16)
                ra[r, sl] = ra[r, sl] + rb[r, sl]
            return 0

        lax.fori_loop(0, CC, row, 0)
        pltpu.sync_copy(ra, out_hbm.at[pl.ds(base + cb, CC)])
        return 0

    lax.fori_loop(0, TW // CC, chunk, 0)


def _combine(ys, pos_a, pos_b):
    mesh = plsc.VectorSubcoreMesh(core_axis_name="c", subcore_axis_name="s")
    f = functools.partial(
        pl.kernel,
        out_type=jax.ShapeDtypeStruct((T, H), jnp.float32),
        mesh=mesh,
        scratch_types=[
            pltpu.VMEM((TW,), jnp.int32),
            pltpu.VMEM((TW,), jnp.int32),
            pltpu.VMEM((CC, H), jnp.float32),
            pltpu.VMEM((CC, H), jnp.float32),
            pltpu.SemaphoreType.DMA,
        ],
    )(_combine_body)
    return f(ys, pos_a, pos_b)


def kernel(hidden_states, gate_w, w1, w2, w3):
    bsz, seq, hd = hidden_states.shape
    x = hidden_states.reshape(-1, hd)

    logits, ti, tw = _router(x, gate_w)

    ef = ti.reshape(-1)
    wf = tw.reshape(-1)
    oh = (ef[:, None] == jnp.arange(E, dtype=jnp.int32)[None, :]).astype(jnp.int32)
    counts = oh.sum(axis=0)
    rank = jnp.take_along_axis(jnp.cumsum(oh, axis=0) - 1,
                               ef[:, None].astype(jnp.int32), axis=1)[:, 0]
    padded = ((counts + BLK - 1) // BLK) * BLK
    offs = jnp.concatenate(
        [jnp.zeros(1, jnp.int32), jnp.cumsum(padded).astype(jnp.int32)])[:E]
    pos = (offs[ef] + rank).astype(jnp.int32)
    tok = (jnp.arange(2 * T, dtype=jnp.int32) // 2)
    row_ids = jnp.zeros((P,), jnp.int32).at[pos].set(tok)
    wrow = jnp.zeros((P,), jnp.float32).at[pos].set(wf)
    tile_start = jnp.arange(NT, dtype=jnp.int32) * BLK
    te = jnp.clip(jnp.searchsorted(offs, tile_start, side="right") - 1,
                  0, E - 1).astype(jnp.int32)

    xs = _gather(x, row_ids)
    ys = _mlp(xs, w1.astype(jnp.bfloat16), w3.astype(jnp.bfloat16),
              w2.astype(jnp.bfloat16), wrow.reshape(NT, 1, BLK), te)
    posr = pos.reshape(T, 2)
    final = _combine(ys, posr[:, 0], posr[:, 1])
    return final.reshape(bsz, seq, hd), logits


# speedup vs baseline: 2.2319x; 1.0665x over previous
import functools

import jax
import jax.numpy as jnp
from jax import lax
from jax.experimental import pallas as pl
from jax.experimental.pallas import tpu as pltpu
from jax.experimental.pallas import tpu_sc as plsc

T = 8192
H = 1024
F = 3584
E = 8
BLK = 256
NT = (T * 2 + E * BLK) // BLK
P = NT * BLK
NC = 2
NS = 16
NW = NC * NS
PW = P // NW
GC = 48
TW = T // NW
CC = 32
NJ = 2
FC = F // NJ
RT = 1024


def _router_body(x_ref, g_ref, lg_ref, ti_ref, tw_ref):
    xb = x_ref[...]
    gw = g_ref[...]
    lg = lax.dot_general(xb, gw, (((1,), (1,)), ((), ())),
                         preferred_element_type=jnp.float32)
    lg_ref[...] = lg
    m = jnp.max(lg, axis=1, keepdims=True)
    p = jnp.exp(lg - m)
    p = p / jnp.sum(p, axis=1, keepdims=True)
    i1 = jnp.argmax(p, axis=1).astype(jnp.int32)
    v1 = jnp.max(p, axis=1)
    cols = lax.broadcasted_iota(jnp.int32, p.shape, 1)
    p2 = jnp.where(cols == i1[:, None], -1.0, p)
    i2 = jnp.argmax(p2, axis=1).astype(jnp.int32)
    v2 = jnp.max(p2, axis=1)
    s = v1 + v2
    ti_ref[...] = jnp.stack([i1, i2], axis=1)
    tw_ref[...] = jnp.stack([v1 / s, v2 / s], axis=1)


def _router(x, gate_w):
    return pl.pallas_call(
        _router_body,
        grid=(T // RT,),
        in_specs=[
            pl.BlockSpec((RT, H), lambda i: (i, 0)),
            pl.BlockSpec((E, H), lambda i: (0, 0)),
        ],
        out_specs=[
            pl.BlockSpec((RT, E), lambda i: (i, 0)),
            pl.BlockSpec((RT, 2), lambda i: (i, 0)),
            pl.BlockSpec((RT, 2), lambda i: (i, 0)),
        ],
        out_shape=[
            jax.ShapeDtypeStruct((T, E), jnp.float32),
            jax.ShapeDtypeStruct((T, 2), jnp.int32),
            jax.ShapeDtypeStruct((T, 2), jnp.float32),
        ],
    )(x, gate_w)


def _mlp_body(te_ref, xs_ref, w1_ref, w3_ref, w2_ref, wr_ref, out_ref):
    xb = xs_ref[...].astype(jnp.bfloat16)
    h1 = lax.dot_general(xb, w1_ref[0], (((1,), (1,)), ((), ())),
                         preferred_element_type=jnp.float32)
    h3 = lax.dot_general(xb, w3_ref[0], (((1,), (1,)), ((), ())),
                         preferred_element_type=jnp.float32)
    h = ((h1 * jax.nn.sigmoid(h1)) * h3).astype(jnp.bfloat16)
    y = lax.dot_general(h, w2_ref[0], (((1,), (1,)), ((), ())),
                        preferred_element_type=jnp.float32)
    out_ref[...] = y * wr_ref[0, 0][:, None]


def _mlp(xs, w1, w3, w2, wrow3, te):
    return pl.pallas_call(
        _mlp_body,
        grid_spec=pltpu.PrefetchScalarGridSpec(
            num_scalar_prefetch=1,
            grid=(NT,),
            in_specs=[
                pl.BlockSpec((BLK, H), lambda i, te_r: (i, 0)),
                pl.BlockSpec((1, F, H), lambda i, te_r: (te_r[i], 0, 0)),
                pl.BlockSpec((1, F, H), lambda i, te_r: (te_r[i], 0, 0)),
                pl.BlockSpec((1, H, F), lambda i, te_r: (te_r[i], 0, 0)),
                pl.BlockSpec((1, 1, BLK), lambda i, te_r: (i, 0, 0)),
            ],
            out_specs=pl.BlockSpec((BLK, H), lambda i, te_r: (i, 0)),
        ),
        out_shape=jax.ShapeDtypeStruct((P, H), jnp.float32),
        compiler_params=pltpu.CompilerParams(
            dimension_semantics=("arbitrary",)),
    )(te, xs, w1, w3, w2, wrow3)


def _gather_body(x_hbm, rid_hbm, out_hbm, idx_v, buf0, buf1, gs0, gs1, ss0, ss1):
    wid = lax.axis_index("s") * NC + lax.axis_index("c")
    base = wid * PW
    pltpu.sync_copy(rid_hbm.at[pl.ds(base, PW)], idx_v)
    n = PW // GC
    buf = (buf0, buf1)
    gsem = (gs0, gs1)
    ssem = (ss0, ss1)
    g = [None, None]
    s = [None, None]
    for c in range(n):
        b = c & 1
        if c == 0:
            g[b] = pltpu.async_copy(
                x_hbm.at[idx_v.at[pl.ds(0, GC)]], buf[b], gsem[b])
        g[b].wait()
        if c + 1 < n:
            nb = 1 - b
            if c >= 1:
                s[nb].wait()
            g[nb] = pltpu.async_copy(
                x_hbm.at[idx_v.at[pl.ds((c + 1) * GC, GC)]], buf[nb], gsem[nb])
        s[b] = pltpu.async_copy(
            buf[b], out_hbm.at[pl.ds(base + c * GC, GC)], ssem[b])
    s[(n - 1) & 1].wait()
    s[(n - 2) & 1].wait()


def _gather(x, row_ids):
    mesh = plsc.VectorSubcoreMesh(core_axis_name="c", subcore_axis_name="s")
    f = functools.partial(
        pl.kernel,
        out_type=jax.ShapeDtypeStruct((P, H), jnp.float32),
        mesh=mesh,
        scratch_types=[
            pltpu.VMEM((PW,), jnp.int32),
            pltpu.VMEM((GC, H), jnp.float32),
            pltpu.VMEM((GC, H), jnp.float32),
            pltpu.SemaphoreType.DMA,
            pltpu.SemaphoreType.DMA,
            pltpu.SemaphoreType.DMA,
            pltpu.SemaphoreType.DMA,
        ],
    )(_gather_body)
    return f(x, row_ids)


def _combine_body(ys_hbm, pa_hbm, pb_hbm, out_hbm, pa_v, pb_v, ra, rb, sem):
    wid = lax.axis_index("s") * NC + lax.axis_index("c")
    base = wid * TW
    pltpu.sync_copy(pa_hbm.at[pl.ds(base, TW)], pa_v)
    pltpu.sync_copy(pb_hbm.at[pl.ds(base, TW)], pb_v)

    def chunk(c, _):
        cb = c * CC
        pltpu.async_copy(ys_hbm.at[pa_v.at[pl.ds(cb, CC)]], ra, sem).wait()
        pltpu.async_copy(ys_hbm.at[pb_v.at[pl.ds(cb, CC)]], rb, sem).wait()

        def row(r, _):
            for q in range(H // 16):
                sl = pl.ds(q * 16, 16)
                ra[r, sl] = ra[r, sl] + rb[r, sl]
            return 0

        lax.fori_loop(0, CC, row, 0)
        pltpu.sync_copy(ra, out_hbm.at[pl.ds(base + cb, CC)])
        return 0

    lax.fori_loop(0, TW // CC, chunk, 0)


def _combine(ys, pos_a, pos_b):
    mesh = plsc.VectorSubcoreMesh(core_axis_name="c", subcore_axis_name="s")
    f = functools.partial(
        pl.kernel,
        out_type=jax.ShapeDtypeStruct((T, H), jnp.float32),
        mesh=mesh,
        scratch_types=[
            pltpu.VMEM((TW,), jnp.int32),
            pltpu.VMEM((TW,), jnp.int32),
            pltpu.VMEM((CC, H), jnp.float32),
            pltpu.VMEM((CC, H), jnp.float32),
            pltpu.SemaphoreType.DMA,
        ],
    )(_combine_body)
    return f(ys, pos_a, pos_b)


def kernel(hidden_states, gate_w, w1, w2, w3):
    bsz, seq, hd = hidden_states.shape
    x = hidden_states.reshape(-1, hd)

    logits, ti, tw = _router(x, gate_w)

    ef = ti.reshape(-1)
    wf = tw.reshape(-1)
    oh = (ef[:, None] == jnp.arange(E, dtype=jnp.int32)[None, :]).astype(jnp.int32)
    counts = oh.sum(axis=0)
    rank = jnp.take_along_axis(jnp.cumsum(oh, axis=0) - 1,
                               ef[:, None].astype(jnp.int32), axis=1)[:, 0]
    padded = ((counts + BLK - 1) // BLK) * BLK
    offs = jnp.concatenate(
        [jnp.zeros(1, jnp.int32), jnp.cumsum(padded).astype(jnp.int32)])[:E]
    pos = (offs[ef] + rank).astype(jnp.int32)
    tile_start = jnp.arange(NT, dtype=jnp.int32) * BLK
    te = jnp.clip(jnp.searchsorted(offs, tile_start, side="right") - 1,
                  0, E - 1).astype(jnp.int32)
    perm = jnp.argsort(ef, stable=True).astype(jnp.int32)
    start = jnp.concatenate(
        [jnp.zeros(1, jnp.int32), jnp.cumsum(counts).astype(jnp.int32)])[:E]
    e_row = jnp.repeat(te, BLK)
    r_row = jnp.arange(P, dtype=jnp.int32) - offs[e_row]
    valid = r_row < counts[e_row]
    j_row = jnp.clip(start[e_row] + r_row, 0, 2 * T - 1)
    ent = perm[j_row]
    row_ids = jnp.where(valid, ent >> 1, 0)
    wrow = jnp.where(valid, wf[ent], 0.0)

    xs = _gather(x, row_ids)
    ys = _mlp(xs, w1.astype(jnp.bfloat16), w3.astype(jnp.bfloat16),
              w2.astype(jnp.bfloat16), wrow.reshape(NT, 1, BLK), te)
    posr = pos.reshape(T, 2)
    final = _combine(ys, posr[:, 0], posr[:, 1])
    return final.reshape(bsz, seq, hd), logits
